# baseline (speedup 1.0000x reference)
import functools

import jax
import jax.numpy as jnp
from jax import lax
from jax.experimental import pallas as pl
from jax.experimental.pallas import tpu as pltpu
from jax.experimental.pallas import tpu_sc as plsc

VOCAB = 1000000
D = 64
SEQ = 200
BATCH = 4096
ROWS = BATCH * SEQ
NW = 32
GBLK = 80
NG = 5
CROWS = GBLK * NG
PER_W = ROWS // NW
NCHUNK = PER_W // CROWS

_mesh = plsc.VectorSubcoreMesh(core_axis_name="c", subcore_axis_name="s")


@functools.partial(
    pl.kernel,
    mesh=_mesh,
    compiler_params=pltpu.CompilerParams(use_tc_tiling_on_sc=False),
    out_type=jax.ShapeDtypeStruct((ROWS, D), jnp.float32),
    scratch_types=[
        pltpu.VMEM((CROWS,), jnp.int32),
        pltpu.VMEM((CROWS, D), jnp.float32),
        pltpu.VMEM((CROWS, D), jnp.float32),
        pltpu.SemaphoreType.DMA,
    ],
)
def _emb_kernel(ids_hbm, tok_hbm, pos_hbm, out_hbm, idx_v, rows_v, pos_v, sem):
    wid = lax.axis_index("s") * 2 + lax.axis_index("c")
    pltpu.sync_copy(pos_hbm, pos_v)
    base0 = wid * PER_W

    def chunk_body(k, carry):
        base = base0 + k * CROWS
        pltpu.sync_copy(ids_hbm.at[pl.ds(base, CROWS)], idx_v)
        copies = [
            pltpu.async_copy(
                tok_hbm.at[idx_v.at[pl.ds(q * GBLK, GBLK)]],
                rows_v.at[pl.ds(q * GBLK, GBLK)],
                sem,
            )
            for q in range(NG)
        ]
        for c in copies:
            c.wait()

        def add_pos(r, c2):
            for c in range(D // 16):
                sl = pl.ds(c * 16, 16)
                rows_v[r, sl] = rows_v[r, sl] + pos_v[r, sl]
            return c2

        lax.fori_loop(0, CROWS, add_pos, 0)
        pltpu.sync_copy(rows_v, out_hbm.at[pl.ds(base, CROWS)])
        return carry

    lax.fori_loop(0, NCHUNK, chunk_body, 0)


def kernel(input_ids, token_emb, pos_emb):
    ids = input_ids.astype(jnp.int32).reshape(ROWS)
    pos_rep = jnp.tile(pos_emb, (CROWS // SEQ, 1))
    out = _emb_kernel(ids, token_emb, pos_rep)
    return out.reshape(BATCH, SEQ, D)


# speedup vs baseline: 1.1147x; 1.1147x over previous
import functools

import jax
import jax.numpy as jnp
from jax import lax
from jax.experimental import pallas as pl
from jax.experimental.pallas import tpu as pltpu
from jax.experimental.pallas import tpu_sc as plsc

VOCAB = 1000000
D = 64
SEQ = 200
BATCH = 4096
ROWS = BATCH * SEQ
NW = 32
GBLK = 80
NG = 5
CROWS = GBLK * NG
PER_W = ROWS // NW
NCHUNK = PER_W // CROWS
UR = 8

_mesh = plsc.VectorSubcoreMesh(core_axis_name="c", subcore_axis_name="s")


@functools.partial(
    pl.kernel,
    mesh=_mesh,
    compiler_params=pltpu.CompilerParams(use_tc_tiling_on_sc=False),
    out_type=jax.ShapeDtypeStruct((ROWS, D), jnp.float32),
    scratch_types=[
        pltpu.VMEM((PER_W,), jnp.int32),
        pltpu.VMEM((2, CROWS, D), jnp.float32),
        pltpu.VMEM((CROWS, D), jnp.float32),
        pltpu.SemaphoreType.DMA,
        pltpu.SemaphoreType.DMA,
        pltpu.SemaphoreType.DMA,
        pltpu.SemaphoreType.DMA,
    ],
)
def _emb_kernel(ids_hbm, tok_hbm, pos_hbm, out_hbm, idx_v, rows_v, pos_v,
                sg0, sg1, sw0, sw1):
    wid = lax.axis_index("s") * 2 + lax.axis_index("c")
    base0 = wid * PER_W
    pltpu.sync_copy(pos_hbm, pos_v)
    pltpu.sync_copy(ids_hbm.at[pl.ds(base0, PER_W)], idx_v)
    sg = (sg0, sg1)
    sw = (sw0, sw1)

    def fire_gathers(k, b):
        for q in range(NG):
            pltpu.async_copy(
                tok_hbm.at[idx_v.at[pl.ds(k * CROWS + q * GBLK, GBLK)]],
                rows_v.at[b, pl.ds(q * GBLK, GBLK)],
                sg[b],
            )

    def drain_gathers(b):
        pltpu.make_async_copy(
            out_hbm.at[pl.ds(base0, CROWS)], rows_v.at[b], sg[b]
        ).wait()

    def drain_writeback(b):
        pltpu.make_async_copy(
            rows_v.at[b], out_hbm.at[pl.ds(base0, CROWS)], sw[b]
        ).wait()

    def add_pos(b):
        def body(r8, c2):
            r0 = r8 * UR
            for dr in range(UR):
                r = r0 + dr
                for c in range(D // 16):
                    sl = pl.ds(c * 16, 16)
                    rows_v[b, r, sl] = rows_v[b, r, sl] + pos_v[r, sl]
            return c2

        lax.fori_loop(0, CROWS // UR, body, 0)

    fire_gathers(0, 0)

    def outer(i, carry):
        for b in range(2):
            k = 2 * i + b
            nb = 1 - b

            @pl.when(k + 1 < NCHUNK)
            def _():
                @pl.when(k >= 1)
                def _():
                    drain_writeback(nb)

                fire_gathers(k + 1, nb)

            drain_gathers(b)
            add_pos(b)
            pltpu.async_copy(
                rows_v.at[b],
                out_hbm.at[pl.ds(base0 + k * CROWS, CROWS)],
                sw[b],
            )
        return carry

    lax.fori_loop(0, NCHUNK // 2, outer, 0)
    drain_writeback(0)
    drain_writeback(1)


def kernel(input_ids, token_emb, pos_emb):
    ids = input_ids.astype(jnp.int32).reshape(ROWS)
    pos_rep = jnp.tile(pos_emb, (CROWS // SEQ, 1))
    out = _emb_kernel(ids, token_emb, pos_rep)
    return out.reshape(BATCH, SEQ, D)


# speedup vs baseline: 1.1184x; 1.0033x over previous
import functools

import jax
import jax.numpy as jnp
from jax import lax
from jax.experimental import pallas as pl
from jax.experimental.pallas import tpu as pltpu
from jax.experimental.pallas import tpu_sc as plsc

VOCAB = 1000000
D = 64
SEQ = 200
BATCH = 4096
ROWS = BATCH * SEQ
NW = 32
GBLK = 80
NG = 5
CROWS = GBLK * NG
PER_W = ROWS // NW
NCHUNK = PER_W // CROWS
DEPTH = 3
UR = 8

_mesh = plsc.VectorSubcoreMesh(core_axis_name="c", subcore_axis_name="s")


@functools.partial(
    pl.kernel,
    mesh=_mesh,
    compiler_params=pltpu.CompilerParams(use_tc_tiling_on_sc=False),
    out_type=jax.ShapeDtypeStruct((ROWS, D), jnp.float32),
    scratch_types=[
        pltpu.VMEM((PER_W,), jnp.int32),
        pltpu.VMEM((DEPTH, CROWS, D), jnp.float32),
        pltpu.VMEM((CROWS, D), jnp.float32),
        pltpu.SemaphoreType.DMA,
        pltpu.SemaphoreType.DMA,
        pltpu.SemaphoreType.DMA,
        pltpu.SemaphoreType.DMA,
        pltpu.SemaphoreType.DMA,
        pltpu.SemaphoreType.DMA,
    ],
)
def _emb_kernel(ids_hbm, tok_hbm, pos_hbm, out_hbm, idx_v, rows_v, pos_v,
                sg0, sg1, sg2, sw0, sw1, sw2):
    wid = lax.axis_index("s") * 2 + lax.axis_index("c")
    base0 = wid * PER_W
    pltpu.sync_copy(pos_hbm, pos_v)
    pltpu.sync_copy(ids_hbm.at[pl.ds(base0, PER_W)], idx_v)
    sg = (sg0, sg1, sg2)
    sw = (sw0, sw1, sw2)

    def fire_gathers(k, b):
        for q in range(NG):
            pltpu.async_copy(
                tok_hbm.at[idx_v.at[pl.ds(k * CROWS + q * GBLK, GBLK)]],
                rows_v.at[b, pl.ds(q * GBLK, GBLK)],
                sg[b],
            )

    def drain_gathers(b):
        pltpu.make_async_copy(
            out_hbm.at[pl.ds(base0, CROWS)], rows_v.at[b], sg[b]
        ).wait()

    def drain_writeback(b):
        pltpu.make_async_copy(
            rows_v.at[b], out_hbm.at[pl.ds(base0, CROWS)], sw[b]
        ).wait()

    def add_pos(b):
        def body(r8, c2):
            r0 = r8 * UR
            for dr in range(UR):
                r = r0 + dr
                for c in range(D // 16):
                    sl = pl.ds(c * 16, 16)
                    rows_v[b, r, sl] = rows_v[b, r, sl] + pos_v[r, sl]
            return c2

        lax.fori_loop(0, CROWS // UR, body, 0)

    for j in range(DEPTH - 1):
        fire_gathers(j, j)

    def outer(i, carry):
        for b in range(DEPTH):
            k = DEPTH * i + b
            fb = (b + DEPTH - 1) % DEPTH

            @pl.when(k + DEPTH - 1 < NCHUNK)
            def _():
                @pl.when(k >= 1)
                def _():
                    drain_writeback(fb)

                fire_gathers(k + DEPTH - 1, fb)

            drain_gathers(b)
            add_pos(b)
            pltpu.async_copy(
                rows_v.at[b],
                out_hbm.at[pl.ds(base0 + k * CROWS, CROWS)],
                sw[b],
            )
        return carry

    lax.fori_loop(0, (NCHUNK - 1) // DEPTH, outer, 0)
    kl = NCHUNK - 1
    bl = kl % DEPTH
    drain_gathers(bl)
    add_pos(bl)
    pltpu.async_copy(
        rows_v.at[bl],
        out_hbm.at[pl.ds(base0 + kl * CROWS, CROWS)],
        sw[bl],
    )
    for b in range(DEPTH):
        drain_writeback(b)


def kernel(input_ids, token_emb, pos_emb):
    ids = input_ids.astype(jnp.int32).reshape(ROWS)
    pos_rep = jnp.tile(pos_emb, (CROWS // SEQ, 1))
    out = _emb_kernel(ids, token_emb, pos_rep)
    return out.reshape(BATCH, SEQ, D)
